# baseline (speedup 1.0000x reference)
import functools

import jax
import jax.numpy as jnp
from jax import lax
from jax.experimental import pallas as pl
from jax.experimental.pallas import tpu as pltpu
from jax.experimental.pallas import tpu_sc as plsc

_NUM_CORES = 2
_NUM_SUBCORES = 16
_NUM_WORKERS = _NUM_CORES * _NUM_SUBCORES

_CHUNK = 64
_NB = 4
_NSET = 2


def _make_gather(N, D):
    per_w = N // _NUM_WORKERS
    n_chunks = per_w // _CHUNK
    n_super = n_chunks // _NB
    assert per_w * _NUM_WORKERS == N
    assert n_chunks * _CHUNK == per_w
    assert n_super * _NB == n_chunks and n_super % _NSET == 0

    mesh = plsc.VectorSubcoreMesh(core_axis_name="c", subcore_axis_name="s")

    nslot = _NSET * _NB
    rows_t = [pltpu.VMEM((_CHUNK, D), jnp.float32) for _ in range(nslot)]
    sems_t = [pltpu.SemaphoreType.DMA for _ in range(2 * nslot)]

    @functools.partial(
        pl.kernel,
        out_type=jax.ShapeDtypeStruct((N, D), jnp.float32),
        mesh=mesh,
        scratch_types=[pltpu.VMEM((n_chunks, _CHUNK), jnp.int32)]
        + rows_t
        + sems_t,
    )
    def gather(emb_hbm, pe_hbm, idx_hbm, out_hbm, idx_v, *bufs):
        rows = [bufs[p * _NB : (p + 1) * _NB] for p in range(_NSET)]
        gsem = [
            bufs[nslot + p * _NB : nslot + (p + 1) * _NB]
            for p in range(_NSET)
        ]
        ssem = [
            bufs[2 * nslot + p * _NB : 2 * nslot + (p + 1) * _NB]
            for p in range(_NSET)
        ]

        wid = lax.axis_index("s") * _NUM_CORES + lax.axis_index("c")
        base = wid * per_w

        def wait_gather(p, b, c):
            pltpu.make_async_copy(
                emb_hbm.at[idx_v.at[c]], rows[p][b], gsem[p][b]
            ).wait()

        def start_store(p, b, c):
            pltpu.async_copy(
                rows[p][b],
                out_hbm.at[pl.ds(base + c * _CHUNK, _CHUNK)],
                ssem[p][b],
            )

        def wait_store(p, b):
            pltpu.make_async_copy(
                rows[p][b], out_hbm.at[pl.ds(base, _CHUNK)], ssem[p][b]
            ).wait()

        def start_gather_emb(p, b, c):
            pltpu.async_copy(emb_hbm.at[idx_v.at[c]], rows[p][b], gsem[p][b])

        def start_gather_pe_add(p, b, c):
            pltpu.async_copy(
                pe_hbm.at[idx_v.at[c]], rows[p][b], gsem[p][b], add=True
            )

        def fill_set(p, c0):
            for b in range(_NB):
                start_gather_emb(p, b, c0 + b)
            for b in range(_NB):
                wait_gather(p, b, c0 + b)
                start_gather_pe_add(p, b, c0 + b)

        pltpu.sync_copy(idx_hbm.at[pl.ds(wid * n_chunks, n_chunks)], idx_v)

        fill_set(0, 0)

        def superstep(t, p):
            c0 = t * _NB
            for b in range(_NB):
                wait_gather(p, b, c0 + b)
                start_store(p, b, c0 + b)
            q = 1 - p
            for b in range(_NB):
                @pl.when(t >= 1)
                def _():
                    wait_store(q, b)
                start_gather_emb(q, b, c0 + _NB + b)
            for b in range(_NB):
                wait_gather(q, b, c0 + _NB + b)
                start_gather_pe_add(q, b, c0 + _NB + b)

        @pl.loop(0, n_super, step=_NSET)
        def _(t0):
            superstep(t0, 0)

            @pl.when(t0 + 2 < n_super)
            def _():
                superstep(t0 + 1, 1)

            @pl.when(t0 + 2 >= n_super)
            def _():
                c0 = (t0 + 1) * _NB
                for b in range(_NB):
                    wait_gather(1, b, c0 + b)
                    start_store(1, b, c0 + b)

        for p in range(_NSET):
            for b in range(_NB):
                wait_store(p, b)

    return gather


def kernel(x, W_emb, W_pe):
    B, L = x.shape
    V, D = W_emb.shape
    N = B * L
    idx2d = x.reshape(N // _CHUNK, _CHUNK).astype(jnp.int32)
    out = _make_gather(N, D)(W_emb, W_pe, idx2d)
    return out.reshape(B, L, D)


# speedup vs baseline: 1.2823x; 1.2823x over previous
import functools

import jax
import jax.numpy as jnp
from jax import lax
from jax.experimental import pallas as pl
from jax.experimental.pallas import tpu as pltpu
from jax.experimental.pallas import tpu_sc as plsc

_NUM_CORES = 2
_NUM_SUBCORES = 16
_NUM_WORKERS = _NUM_CORES * _NUM_SUBCORES

_CHUNK = 128
_NB = 2
_NSET = 2


def _add_body(a_ref, b_ref, o_ref):
    o_ref[...] = a_ref[...] + b_ref[...]


def _sum_tables(W_emb, W_pe):
    V, D = W_emb.shape
    rb = 4000
    assert V % rb == 0
    return pl.pallas_call(
        _add_body,
        grid=(V // rb,),
        in_specs=[
            pl.BlockSpec((rb, D), lambda i: (i, 0)),
            pl.BlockSpec((rb, D), lambda i: (i, 0)),
        ],
        out_specs=pl.BlockSpec((rb, D), lambda i: (i, 0)),
        out_shape=jax.ShapeDtypeStruct((V, D), jnp.float32),
    )(W_emb, W_pe)


def _make_gather(N, D):
    per_w = N // _NUM_WORKERS
    n_chunks = per_w // _CHUNK
    n_super = n_chunks // _NB
    assert per_w * _NUM_WORKERS == N
    assert n_chunks * _CHUNK == per_w
    assert n_super * _NB == n_chunks and n_super % _NSET == 0

    mesh = plsc.VectorSubcoreMesh(core_axis_name="c", subcore_axis_name="s")

    nslot = _NSET * _NB
    rows_t = [pltpu.VMEM((_CHUNK, D), jnp.float32) for _ in range(nslot)]
    sems_t = [pltpu.SemaphoreType.DMA for _ in range(2 * nslot)]

    @functools.partial(
        pl.kernel,
        out_type=jax.ShapeDtypeStruct((N, D), jnp.float32),
        mesh=mesh,
        scratch_types=[pltpu.VMEM((n_chunks, _CHUNK), jnp.int32)]
        + rows_t
        + sems_t,
    )
    def gather(table_hbm, idx_hbm, out_hbm, idx_v, *bufs):
        rows = [bufs[p * _NB : (p + 1) * _NB] for p in range(_NSET)]
        gsem = [
            bufs[nslot + p * _NB : nslot + (p + 1) * _NB]
            for p in range(_NSET)
        ]
        ssem = [
            bufs[2 * nslot + p * _NB : 2 * nslot + (p + 1) * _NB]
            for p in range(_NSET)
        ]

        wid = lax.axis_index("s") * _NUM_CORES + lax.axis_index("c")
        base = wid * per_w

        def wait_gather(p, b, c):
            pltpu.make_async_copy(
                table_hbm.at[idx_v.at[c]], rows[p][b], gsem[p][b]
            ).wait()

        def start_store(p, b, c):
            pltpu.async_copy(
                rows[p][b],
                out_hbm.at[pl.ds(base + c * _CHUNK, _CHUNK)],
                ssem[p][b],
            )

        def wait_store(p, b):
            pltpu.make_async_copy(
                rows[p][b], out_hbm.at[pl.ds(base, _CHUNK)], ssem[p][b]
            ).wait()

        def start_gather(p, b, c):
            pltpu.async_copy(table_hbm.at[idx_v.at[c]], rows[p][b], gsem[p][b])

        pltpu.sync_copy(idx_hbm.at[pl.ds(wid * n_chunks, n_chunks)], idx_v)

        for b in range(_NB):
            start_gather(0, b, b)

        def superstep(t, p):
            c0 = t * _NB
            for b in range(_NB):
                wait_gather(p, b, c0 + b)
                start_store(p, b, c0 + b)
            q = 1 - p
            for b in range(_NB):
                @pl.when(t >= 1)
                def _():
                    wait_store(q, b)
                start_gather(q, b, c0 + _NB + b)

        @pl.loop(0, n_super, step=_NSET)
        def _(t0):
            superstep(t0, 0)

            @pl.when(t0 + 2 < n_super)
            def _():
                superstep(t0 + 1, 1)

            @pl.when(t0 + 2 >= n_super)
            def _():
                c0 = (t0 + 1) * _NB
                for b in range(_NB):
                    wait_gather(1, b, c0 + b)
                    start_store(1, b, c0 + b)

        for p in range(_NSET):
            for b in range(_NB):
                wait_store(p, b)

    return gather


def kernel(x, W_emb, W_pe):
    B, L = x.shape
    V, D = W_emb.shape
    N = B * L
    W_sum = _sum_tables(W_emb, W_pe)
    idx2d = x.reshape(N // _CHUNK, _CHUNK).astype(jnp.int32)
    out = _make_gather(N, D)(W_sum, idx2d)
    return out.reshape(B, L, D)


# speedup vs baseline: 1.2916x; 1.0073x over previous
import functools

import jax
import jax.numpy as jnp
from jax import lax
from jax.experimental import pallas as pl
from jax.experimental.pallas import tpu as pltpu
from jax.experimental.pallas import tpu_sc as plsc

_NUM_CORES = 2
_NUM_SUBCORES = 16
_NUM_WORKERS = _NUM_CORES * _NUM_SUBCORES

_CHUNK = 64
_NB = 4
_NSET = 2


def _add_body(a_ref, b_ref, o_ref):
    o_ref[...] = a_ref[...] + b_ref[...]


def _sum_tables(W_emb, W_pe):
    V, D = W_emb.shape
    rb = 4000
    assert V % rb == 0
    return pl.pallas_call(
        _add_body,
        grid=(V // rb,),
        in_specs=[
            pl.BlockSpec((rb, D), lambda i: (i, 0)),
            pl.BlockSpec((rb, D), lambda i: (i, 0)),
        ],
        out_specs=pl.BlockSpec((rb, D), lambda i: (i, 0)),
        out_shape=jax.ShapeDtypeStruct((V, D), jnp.float32),
    )(W_emb, W_pe)


def _make_gather(N, D):
    per_w = N // _NUM_WORKERS
    n_chunks = per_w // _CHUNK
    n_super = n_chunks // _NB
    assert per_w * _NUM_WORKERS == N
    assert n_chunks * _CHUNK == per_w
    assert n_super * _NB == n_chunks and n_super % _NSET == 0

    mesh = plsc.VectorSubcoreMesh(core_axis_name="c", subcore_axis_name="s")

    nslot = _NSET * _NB
    rows_t = [pltpu.VMEM((_CHUNK, D), jnp.float32) for _ in range(nslot)]
    sems_t = [pltpu.SemaphoreType.DMA for _ in range(2 * nslot)]

    @functools.partial(
        pl.kernel,
        out_type=jax.ShapeDtypeStruct((N, D), jnp.float32),
        mesh=mesh,
        scratch_types=[pltpu.VMEM((n_chunks, _CHUNK), jnp.int32)]
        + rows_t
        + sems_t,
    )
    def gather(table_hbm, idx_hbm, out_hbm, idx_v, *bufs):
        rows = [bufs[p * _NB : (p + 1) * _NB] for p in range(_NSET)]
        gsem = [
            bufs[nslot + p * _NB : nslot + (p + 1) * _NB]
            for p in range(_NSET)
        ]
        ssem = [
            bufs[2 * nslot + p * _NB : 2 * nslot + (p + 1) * _NB]
            for p in range(_NSET)
        ]

        wid = lax.axis_index("s") * _NUM_CORES + lax.axis_index("c")
        base = wid * per_w

        def wait_gather(p, b, c):
            pltpu.make_async_copy(
                table_hbm.at[idx_v.at[c]], rows[p][b], gsem[p][b]
            ).wait()

        def start_store(p, b, c):
            pltpu.async_copy(
                rows[p][b],
                out_hbm.at[pl.ds(base + c * _CHUNK, _CHUNK)],
                ssem[p][b],
            )

        def wait_store(p, b):
            pltpu.make_async_copy(
                rows[p][b], out_hbm.at[pl.ds(base, _CHUNK)], ssem[p][b]
            ).wait()

        def start_gather(p, b, c):
            pltpu.async_copy(table_hbm.at[idx_v.at[c]], rows[p][b], gsem[p][b])

        pltpu.sync_copy(idx_hbm.at[pl.ds(wid * n_chunks, n_chunks)], idx_v)

        for b in range(_NB):
            start_gather(0, b, b)

        def superstep(t, p):
            c0 = t * _NB
            q = 1 - p
            for b in range(_NB):
                wait_gather(p, b, c0 + b)
                start_store(p, b, c0 + b)

                @pl.when(t >= 1)
                def _():
                    wait_store(q, b)
                start_gather(q, b, c0 + _NB + b)

        @pl.loop(0, n_super, step=_NSET)
        def _(t0):
            superstep(t0, 0)

            @pl.when(t0 + 2 < n_super)
            def _():
                superstep(t0 + 1, 1)

            @pl.when(t0 + 2 >= n_super)
            def _():
                c0 = (t0 + 1) * _NB
                for b in range(_NB):
                    wait_gather(1, b, c0 + b)
                    start_store(1, b, c0 + b)

        for p in range(_NSET):
            for b in range(_NB):
                wait_store(p, b)

    return gather


def kernel(x, W_emb, W_pe):
    B, L = x.shape
    V, D = W_emb.shape
    N = B * L
    W_sum = _sum_tables(W_emb, W_pe)
    idx2d = x.reshape(N // _CHUNK, _CHUNK).astype(jnp.int32)
    out = _make_gather(N, D)(W_sum, idx2d)
    return out.reshape(B, L, D)
